# Initial kernel scaffold; baseline (speedup 1.0000x reference)
import jax
import jax.numpy as jnp
from jax.experimental import pallas as pl


def kernel(x):
    raise NotImplementedError("write your pallas kernel here")


# speedup vs baseline: 2.4980x; 2.4980x over previous
import jax
import jax.numpy as jnp
from jax import lax
from jax.experimental import pallas as pl
from jax.experimental.pallas import tpu as pltpu
from jax.experimental.pallas import tpu_sc as plsc

_N = 8192
_D = 128
_EPS = 0.4
_BI = 512
_BJ = 1024
_BS = 512
_NW = 32
_CHUNK = _N // _NW


def _build_mask_body(xi_ref, xj_ref, mask_ref, nbr_ref):
    j = pl.program_id(1)
    xi = xi_ref[...]
    xj = xj_ref[...]
    xi = xi / jnp.sqrt(jnp.sum(xi * xi, axis=1, keepdims=True))
    xj = xj / jnp.sqrt(jnp.sum(xj * xj, axis=1, keepdims=True))
    s = lax.dot_general(xi, xj, (((1,), (1,)), ((), ())),
                        preferred_element_type=jnp.float32,
                        precision=lax.Precision.HIGHEST)
    m = s > _EPS
    mask_ref[...] = m.astype(jnp.int8)
    jidx = lax.broadcasted_iota(jnp.int32, (_BI, _BJ), 1) + j * _BJ
    pmin = jnp.min(jnp.where(m, jidx, _N), axis=1).reshape(1, _BI)

    @pl.when(j == 0)
    def _():
        nbr_ref[0, :, :] = pmin

    @pl.when(j != 0)
    def _():
        nbr_ref[0, :, :] = jnp.minimum(nbr_ref[0, :, :], pmin)


def _build_mask(x):
    return pl.pallas_call(
        _build_mask_body,
        grid=(_N // _BI, _N // _BJ),
        in_specs=[pl.BlockSpec((_BI, _D), lambda i, j: (i, 0)),
                  pl.BlockSpec((_BJ, _D), lambda i, j: (j, 0))],
        out_specs=[pl.BlockSpec((_BI, _BJ), lambda i, j: (i, j)),
                   pl.BlockSpec((1, 1, _BI), lambda i, j: (i, 0, 0))],
        out_shape=[jax.ShapeDtypeStruct((_N, _N), jnp.int8),
                   jax.ShapeDtypeStruct((_N // _BI, 1, _BI), jnp.int32)],
    )(x, x)


def _sweep_body(mask_ref, comp_ref, out_ref):
    m = mask_ref[...].astype(jnp.int32) != 0
    c = jnp.broadcast_to(comp_ref[...], (_BS, _N))
    out_ref[0, :, :] = jnp.min(jnp.where(m, c, _N), axis=1).reshape(1, _BS)


def _sweep(mask, comp):
    out = pl.pallas_call(
        _sweep_body,
        grid=(_N // _BS,),
        in_specs=[pl.BlockSpec((_BS, _N), lambda i: (i, 0)),
                  pl.BlockSpec((1, _N), lambda i: (0, 0))],
        out_specs=pl.BlockSpec((1, 1, _BS), lambda i: (i, 0, 0)),
        out_shape=jax.ShapeDtypeStruct((_N // _BS, 1, _BS), jnp.int32),
    )(mask, comp.reshape(1, _N))
    return out.reshape(_N)


_SC_MESH = plsc.VectorSubcoreMesh(core_axis_name="c", subcore_axis_name="s")


def _jump_body(comp_hbm, out_hbm, comp_v, idx_v, gat_v, out_v, sem):
    wid = lax.axis_index("s") * 2 + lax.axis_index("c")
    base = wid * _CHUNK
    pltpu.sync_copy(comp_hbm.at[pl.ds(base, _CHUNK)], comp_v)
    cap = jnp.full((16,), _N - 1, jnp.int32)

    def clampstep(k, carry):
        idx_v[pl.ds(k * 16, 16)] = jnp.minimum(comp_v[pl.ds(k * 16, 16)], cap)
        return carry

    lax.fori_loop(0, _CHUNK // 16, clampstep, 0)
    cps = [pltpu.async_copy(comp_hbm.at[idx_v.at[pl.ds(j * 128, 128)]],
                            gat_v.at[pl.ds(j * 128, 128)], sem)
           for j in range(_CHUNK // 128)]
    for cp in cps:
        cp.wait()

    def minstep(k, carry):
        out_v[pl.ds(k * 16, 16)] = jnp.minimum(comp_v[pl.ds(k * 16, 16)],
                                               gat_v[pl.ds(k * 16, 16)])
        return carry

    lax.fori_loop(0, _CHUNK // 16, minstep, 0)
    pltpu.sync_copy(out_v, out_hbm.at[pl.ds(base, _CHUNK)])


_jump = pl.kernel(
    _jump_body,
    mesh=_SC_MESH,
    out_type=jax.ShapeDtypeStruct((_N,), jnp.int32),
    scratch_types=[pltpu.VMEM((_CHUNK,), jnp.int32),
                   pltpu.VMEM((_CHUNK,), jnp.int32),
                   pltpu.VMEM((_CHUNK,), jnp.int32),
                   pltpu.VMEM((_CHUNK,), jnp.int32),
                   pltpu.SemaphoreType.DMA],
)


def _rank_body(comp_ref, rank_ref):
    c = comp_ref[...]
    gidx = (lax.broadcasted_iota(jnp.int32, (64, 128), 0) * 128
            + lax.broadcasted_iota(jnp.int32, (64, 128), 1))
    isr = (c == gidx).astype(jnp.float32)
    tri = (lax.broadcasted_iota(jnp.int32, (128, 128), 0)
           <= lax.broadcasted_iota(jnp.int32, (128, 128), 1)
           ).astype(jnp.float32)
    rowcs = lax.dot_general(isr, tri, (((1,), (0,)), ((), ())),
                            preferred_element_type=jnp.float32,
                            precision=lax.Precision.HIGHEST)
    stri = (lax.broadcasted_iota(jnp.int32, (64, 64), 1)
            < lax.broadcasted_iota(jnp.int32, (64, 64), 0)
            ).astype(jnp.float32)
    off = lax.dot_general(stri, rowcs[:, 127:128], (((1,), (0,)), ((), ())),
                          preferred_element_type=jnp.float32,
                          precision=lax.Precision.HIGHEST)
    rank_ref[...] = (rowcs + off - 1.0).astype(jnp.int32)


def _rank(comp):
    out = pl.pallas_call(
        _rank_body,
        out_shape=jax.ShapeDtypeStruct((64, 128), jnp.int32),
    )(comp.reshape(64, 128))
    return out.reshape(_N)


def _label_body(comp_hbm, rank_hbm, nbr_hbm, out_hbm,
                comp_v, nbr_v, idx_v, gat_v, out_v, sem):
    wid = lax.axis_index("s") * 2 + lax.axis_index("c")
    base = wid * _CHUNK
    pltpu.sync_copy(comp_hbm.at[pl.ds(base, _CHUNK)], comp_v)
    pltpu.sync_copy(nbr_hbm.at[pl.ds(base, _CHUNK)], nbr_v)
    cap = jnp.full((16,), _N - 1, jnp.int32)

    def clampstep(k, carry):
        idx_v[pl.ds(k * 16, 16)] = jnp.minimum(comp_v[pl.ds(k * 16, 16)], cap)
        return carry

    lax.fori_loop(0, _CHUNK // 16, clampstep, 0)
    cps = [pltpu.async_copy(rank_hbm.at[idx_v.at[pl.ds(j * 128, 128)]],
                            gat_v.at[pl.ds(j * 128, 128)], sem)
           for j in range(_CHUNK // 128)]
    for cp in cps:
        cp.wait()
    nval = jnp.full((16,), _N, jnp.int32)
    neg1 = jnp.full((16,), -1, jnp.int32)

    def selstep(k, carry):
        nb = nbr_v[pl.ds(k * 16, 16)]
        lbl = gat_v[pl.ds(k * 16, 16)]
        out_v[pl.ds(k * 16, 16)] = jnp.where(nb < nval, lbl, neg1)
        return carry

    lax.fori_loop(0, _CHUNK // 16, selstep, 0)
    pltpu.sync_copy(out_v, out_hbm.at[pl.ds(base, _CHUNK)])


_label = pl.kernel(
    _label_body,
    mesh=_SC_MESH,
    out_type=jax.ShapeDtypeStruct((_N,), jnp.int32),
    scratch_types=[pltpu.VMEM((_CHUNK,), jnp.int32),
                   pltpu.VMEM((_CHUNK,), jnp.int32),
                   pltpu.VMEM((_CHUNK,), jnp.int32),
                   pltpu.VMEM((_CHUNK,), jnp.int32),
                   pltpu.VMEM((_CHUNK,), jnp.int32),
                   pltpu.SemaphoreType.DMA],
)


def kernel(x):
    mask, nbr3 = _build_mask(x)
    comp1 = nbr3.reshape(_N)
    comp = _jump(comp1)

    def body(st):
        t, comp, _ = st
        c2 = _jump(_sweep(mask, comp))
        return t + 1, c2, jnp.any(c2 != comp)

    def cond(st):
        return st[2] & (st[0] < 16)

    _, comp, _ = lax.while_loop(cond, body,
                                (jnp.int32(1), comp, jnp.bool_(True)))
    rank = _rank(comp)
    return _label(comp, rank, comp1)


# speedup vs baseline: 2.5909x; 1.0372x over previous
import jax
import jax.numpy as jnp
from jax import lax
from jax.experimental import pallas as pl
from jax.experimental.pallas import tpu as pltpu
from jax.experimental.pallas import tpu_sc as plsc

_N = 8192
_D = 128
_EPS = 0.4
_BI = 512
_BJ = 1024
_BS = 512
_NW = 32
_CHUNK = _N // _NW


def _normalize_body(x_ref, xn_ref):
    x = x_ref[...]
    xn_ref[...] = x / jnp.sqrt(jnp.sum(x * x, axis=1, keepdims=True))


def _normalize(x):
    return pl.pallas_call(
        _normalize_body,
        out_shape=jax.ShapeDtypeStruct((_N, _D), jnp.float32),
    )(x)


def _build_mask_body(xi_ref, xj_ref, mask_ref, nbr_ref):
    j = pl.program_id(1)
    xi = xi_ref[...]
    xj = xj_ref[...]
    s = lax.dot_general(xi, xj, (((1,), (1,)), ((), ())),
                        preferred_element_type=jnp.float32,
                        precision=lax.Precision.HIGHEST)
    m = s > _EPS
    mask_ref[...] = m.astype(jnp.int8)
    jidx = lax.broadcasted_iota(jnp.int32, (_BI, _BJ), 1) + j * _BJ
    pmin = jnp.min(jnp.where(m, jidx, _N), axis=1).reshape(1, _BI)

    @pl.when(j == 0)
    def _():
        nbr_ref[0, :, :] = pmin

    @pl.when(j != 0)
    def _():
        nbr_ref[0, :, :] = jnp.minimum(nbr_ref[0, :, :], pmin)


def _build_mask(x):
    return pl.pallas_call(
        _build_mask_body,
        grid=(_N // _BI, _N // _BJ),
        in_specs=[pl.BlockSpec((_BI, _D), lambda i, j: (i, 0)),
                  pl.BlockSpec((_BJ, _D), lambda i, j: (j, 0))],
        out_specs=[pl.BlockSpec((_BI, _BJ), lambda i, j: (i, j)),
                   pl.BlockSpec((1, 1, _BI), lambda i, j: (i, 0, 0))],
        out_shape=[jax.ShapeDtypeStruct((_N, _N), jnp.int8),
                   jax.ShapeDtypeStruct((_N // _BI, 1, _BI), jnp.int32)],
    )(x, x)


def _sweep_body(mask_ref, comp_ref, out_ref):
    m = mask_ref[...].astype(jnp.int32) != 0
    c = jnp.broadcast_to(comp_ref[...], (_BS, _N))
    out_ref[0, :, :] = jnp.min(jnp.where(m, c, _N), axis=1).reshape(1, _BS)


def _sweep(mask, comp):
    out = pl.pallas_call(
        _sweep_body,
        grid=(_N // _BS,),
        in_specs=[pl.BlockSpec((_BS, _N), lambda i: (i, 0)),
                  pl.BlockSpec((1, _N), lambda i: (0, 0))],
        out_specs=pl.BlockSpec((1, 1, _BS), lambda i: (i, 0, 0)),
        out_shape=jax.ShapeDtypeStruct((_N // _BS, 1, _BS), jnp.int32),
    )(mask, comp.reshape(1, _N))
    return out.reshape(_N)


_SC_MESH = plsc.VectorSubcoreMesh(core_axis_name="c", subcore_axis_name="s")


def _jump_body(comp_hbm, out_hbm, comp_v, idx_v, gat_v, out_v, sem):
    wid = lax.axis_index("s") * 2 + lax.axis_index("c")
    base = wid * _CHUNK
    pltpu.sync_copy(comp_hbm.at[pl.ds(base, _CHUNK)], comp_v)
    cap = jnp.full((16,), _N - 1, jnp.int32)

    def clampstep(k, carry):
        idx_v[pl.ds(k * 16, 16)] = jnp.minimum(comp_v[pl.ds(k * 16, 16)], cap)
        return carry

    lax.fori_loop(0, _CHUNK // 16, clampstep, 0)
    cps = [pltpu.async_copy(comp_hbm.at[idx_v.at[pl.ds(j * 128, 128)]],
                            gat_v.at[pl.ds(j * 128, 128)], sem)
           for j in range(_CHUNK // 128)]
    for cp in cps:
        cp.wait()

    def minstep(k, carry):
        out_v[pl.ds(k * 16, 16)] = jnp.minimum(comp_v[pl.ds(k * 16, 16)],
                                               gat_v[pl.ds(k * 16, 16)])
        return carry

    lax.fori_loop(0, _CHUNK // 16, minstep, 0)
    pltpu.sync_copy(out_v, out_hbm.at[pl.ds(base, _CHUNK)])


_jump = pl.kernel(
    _jump_body,
    mesh=_SC_MESH,
    out_type=jax.ShapeDtypeStruct((_N,), jnp.int32),
    scratch_types=[pltpu.VMEM((_CHUNK,), jnp.int32),
                   pltpu.VMEM((_CHUNK,), jnp.int32),
                   pltpu.VMEM((_CHUNK,), jnp.int32),
                   pltpu.VMEM((_CHUNK,), jnp.int32),
                   pltpu.SemaphoreType.DMA],
)


def _rank_body(comp_ref, rank_ref):
    c = comp_ref[...]
    gidx = (lax.broadcasted_iota(jnp.int32, (64, 128), 0) * 128
            + lax.broadcasted_iota(jnp.int32, (64, 128), 1))
    isr = (c == gidx).astype(jnp.float32)
    tri = (lax.broadcasted_iota(jnp.int32, (128, 128), 0)
           <= lax.broadcasted_iota(jnp.int32, (128, 128), 1)
           ).astype(jnp.float32)
    rowcs = lax.dot_general(isr, tri, (((1,), (0,)), ((), ())),
                            preferred_element_type=jnp.float32,
                            precision=lax.Precision.HIGHEST)
    stri = (lax.broadcasted_iota(jnp.int32, (64, 64), 1)
            < lax.broadcasted_iota(jnp.int32, (64, 64), 0)
            ).astype(jnp.float32)
    off = lax.dot_general(stri, rowcs[:, 127:128], (((1,), (0,)), ((), ())),
                          preferred_element_type=jnp.float32,
                          precision=lax.Precision.HIGHEST)
    rank_ref[...] = (rowcs + off - 1.0).astype(jnp.int32)


def _rank(comp):
    out = pl.pallas_call(
        _rank_body,
        out_shape=jax.ShapeDtypeStruct((64, 128), jnp.int32),
    )(comp.reshape(64, 128))
    return out.reshape(_N)


def _label_body(comp_hbm, rank_hbm, nbr_hbm, out_hbm,
                comp_v, nbr_v, idx_v, gat_v, out_v, sem):
    wid = lax.axis_index("s") * 2 + lax.axis_index("c")
    base = wid * _CHUNK
    pltpu.sync_copy(comp_hbm.at[pl.ds(base, _CHUNK)], comp_v)
    pltpu.sync_copy(nbr_hbm.at[pl.ds(base, _CHUNK)], nbr_v)
    cap = jnp.full((16,), _N - 1, jnp.int32)

    def clampstep(k, carry):
        idx_v[pl.ds(k * 16, 16)] = jnp.minimum(comp_v[pl.ds(k * 16, 16)], cap)
        return carry

    lax.fori_loop(0, _CHUNK // 16, clampstep, 0)
    cps = [pltpu.async_copy(rank_hbm.at[idx_v.at[pl.ds(j * 128, 128)]],
                            gat_v.at[pl.ds(j * 128, 128)], sem)
           for j in range(_CHUNK // 128)]
    for cp in cps:
        cp.wait()
    nval = jnp.full((16,), _N, jnp.int32)
    neg1 = jnp.full((16,), -1, jnp.int32)

    def selstep(k, carry):
        nb = nbr_v[pl.ds(k * 16, 16)]
        lbl = gat_v[pl.ds(k * 16, 16)]
        out_v[pl.ds(k * 16, 16)] = jnp.where(nb < nval, lbl, neg1)
        return carry

    lax.fori_loop(0, _CHUNK // 16, selstep, 0)
    pltpu.sync_copy(out_v, out_hbm.at[pl.ds(base, _CHUNK)])


_label = pl.kernel(
    _label_body,
    mesh=_SC_MESH,
    out_type=jax.ShapeDtypeStruct((_N,), jnp.int32),
    scratch_types=[pltpu.VMEM((_CHUNK,), jnp.int32),
                   pltpu.VMEM((_CHUNK,), jnp.int32),
                   pltpu.VMEM((_CHUNK,), jnp.int32),
                   pltpu.VMEM((_CHUNK,), jnp.int32),
                   pltpu.VMEM((_CHUNK,), jnp.int32),
                   pltpu.SemaphoreType.DMA],
)


def kernel(x):
    xn = _normalize(x)
    mask, nbr3 = _build_mask(xn)
    comp1 = nbr3.reshape(_N)
    comp = _jump(comp1)

    def body(st):
        t, comp, _ = st
        c2 = _jump(_sweep(mask, comp))
        return t + 1, c2, jnp.any(c2 != comp)

    def cond(st):
        return st[2] & (st[0] < 16)

    _, comp, _ = lax.while_loop(cond, body,
                                (jnp.int32(1), comp, jnp.bool_(True)))
    rank = _rank(comp)
    return _label(comp, rank, comp1)


# speedup vs baseline: 3.4976x; 1.3499x over previous
import jax
import jax.numpy as jnp
from jax import lax
from jax.experimental import pallas as pl
from jax.experimental.pallas import tpu as pltpu
from jax.experimental.pallas import tpu_sc as plsc

_N = 8192
_D = 128
_EPS = 0.4
_BI = 512
_BJ = 1024
_BS = 512
_NW = 32
_CHUNK = _N // _NW


def _normalize_body(x_ref, xh_ref, xl_ref):
    x = x_ref[...]
    xn = x / jnp.sqrt(jnp.sum(x * x, axis=1, keepdims=True))
    hi = xn.astype(jnp.bfloat16)
    xh_ref[...] = hi
    xl_ref[...] = (xn - hi.astype(jnp.float32)).astype(jnp.bfloat16)


def _normalize(x):
    return pl.pallas_call(
        _normalize_body,
        out_shape=[jax.ShapeDtypeStruct((_N, _D), jnp.bfloat16),
                   jax.ShapeDtypeStruct((_N, _D), jnp.bfloat16)],
    )(x)


def _bf16_dot(a, b):
    return lax.dot_general(a, b, (((1,), (1,)), ((), ())),
                           preferred_element_type=jnp.float32)


def _build_mask_body(xhi_ref, xli_ref, xhj_ref, xlj_ref, mask_ref, nbr_ref):
    j = pl.program_id(1)
    xhi = xhi_ref[...]
    xli = xli_ref[...]
    xhj = xhj_ref[...]
    xlj = xlj_ref[...]
    s = _bf16_dot(xhi, xhj) + _bf16_dot(xhi, xlj) + _bf16_dot(xli, xhj)
    m = s > _EPS
    mask_ref[...] = m.astype(jnp.int8)
    jidx = lax.broadcasted_iota(jnp.int32, (_BI, _BJ), 1) + j * _BJ
    pmin = jnp.min(jnp.where(m, jidx, _N), axis=1).reshape(1, _BI)

    @pl.when(j == 0)
    def _():
        nbr_ref[0, :, :] = pmin

    @pl.when(j != 0)
    def _():
        nbr_ref[0, :, :] = jnp.minimum(nbr_ref[0, :, :], pmin)


def _build_mask(xh, xl):
    return pl.pallas_call(
        _build_mask_body,
        grid=(_N // _BI, _N // _BJ),
        in_specs=[pl.BlockSpec((_BI, _D), lambda i, j: (i, 0)),
                  pl.BlockSpec((_BI, _D), lambda i, j: (i, 0)),
                  pl.BlockSpec((_BJ, _D), lambda i, j: (j, 0)),
                  pl.BlockSpec((_BJ, _D), lambda i, j: (j, 0))],
        out_specs=[pl.BlockSpec((_BI, _BJ), lambda i, j: (i, j)),
                   pl.BlockSpec((1, 1, _BI), lambda i, j: (i, 0, 0))],
        out_shape=[jax.ShapeDtypeStruct((_N, _N), jnp.int8),
                   jax.ShapeDtypeStruct((_N // _BI, 1, _BI), jnp.int32)],
    )(xh, xl, xh, xl)


def _sweep_body(mask_ref, comp_ref, out_ref):
    m = mask_ref[...].astype(jnp.int32) != 0
    c = jnp.broadcast_to(comp_ref[...], (_BS, _N))
    out_ref[0, :, :] = jnp.min(jnp.where(m, c, _N), axis=1).reshape(1, _BS)


def _sweep(mask, comp):
    out = pl.pallas_call(
        _sweep_body,
        grid=(_N // _BS,),
        in_specs=[pl.BlockSpec((_BS, _N), lambda i: (i, 0)),
                  pl.BlockSpec((1, _N), lambda i: (0, 0))],
        out_specs=pl.BlockSpec((1, 1, _BS), lambda i: (i, 0, 0)),
        out_shape=jax.ShapeDtypeStruct((_N // _BS, 1, _BS), jnp.int32),
    )(mask, comp.reshape(1, _N))
    return out.reshape(_N)


_SC_MESH = plsc.VectorSubcoreMesh(core_axis_name="c", subcore_axis_name="s")


def _jump_body(comp_hbm, out_hbm, comp_v, idx_v, gat_v, out_v, sem):
    wid = lax.axis_index("s") * 2 + lax.axis_index("c")
    base = wid * _CHUNK
    pltpu.sync_copy(comp_hbm.at[pl.ds(base, _CHUNK)], comp_v)
    cap = jnp.full((16,), _N - 1, jnp.int32)

    def clampstep(k, carry):
        idx_v[pl.ds(k * 16, 16)] = jnp.minimum(comp_v[pl.ds(k * 16, 16)], cap)
        return carry

    lax.fori_loop(0, _CHUNK // 16, clampstep, 0)
    cps = [pltpu.async_copy(comp_hbm.at[idx_v.at[pl.ds(j * 128, 128)]],
                            gat_v.at[pl.ds(j * 128, 128)], sem)
           for j in range(_CHUNK // 128)]
    for cp in cps:
        cp.wait()

    def minstep(k, carry):
        out_v[pl.ds(k * 16, 16)] = jnp.minimum(comp_v[pl.ds(k * 16, 16)],
                                               gat_v[pl.ds(k * 16, 16)])
        return carry

    lax.fori_loop(0, _CHUNK // 16, minstep, 0)
    pltpu.sync_copy(out_v, out_hbm.at[pl.ds(base, _CHUNK)])


_jump = pl.kernel(
    _jump_body,
    mesh=_SC_MESH,
    out_type=jax.ShapeDtypeStruct((_N,), jnp.int32),
    scratch_types=[pltpu.VMEM((_CHUNK,), jnp.int32),
                   pltpu.VMEM((_CHUNK,), jnp.int32),
                   pltpu.VMEM((_CHUNK,), jnp.int32),
                   pltpu.VMEM((_CHUNK,), jnp.int32),
                   pltpu.SemaphoreType.DMA],
)


def _rank_body(comp_ref, rank_ref):
    c = comp_ref[...]
    gidx = (lax.broadcasted_iota(jnp.int32, (64, 128), 0) * 128
            + lax.broadcasted_iota(jnp.int32, (64, 128), 1))
    isr = (c == gidx).astype(jnp.float32)
    tri = (lax.broadcasted_iota(jnp.int32, (128, 128), 0)
           <= lax.broadcasted_iota(jnp.int32, (128, 128), 1)
           ).astype(jnp.float32)
    rowcs = lax.dot_general(isr, tri, (((1,), (0,)), ((), ())),
                            preferred_element_type=jnp.float32,
                            precision=lax.Precision.HIGHEST)
    stri = (lax.broadcasted_iota(jnp.int32, (64, 64), 1)
            < lax.broadcasted_iota(jnp.int32, (64, 64), 0)
            ).astype(jnp.float32)
    off = lax.dot_general(stri, rowcs[:, 127:128], (((1,), (0,)), ((), ())),
                          preferred_element_type=jnp.float32,
                          precision=lax.Precision.HIGHEST)
    rank_ref[...] = (rowcs + off - 1.0).astype(jnp.int32)


def _rank(comp):
    out = pl.pallas_call(
        _rank_body,
        out_shape=jax.ShapeDtypeStruct((64, 128), jnp.int32),
    )(comp.reshape(64, 128))
    return out.reshape(_N)


def _label_body(comp_hbm, rank_hbm, nbr_hbm, out_hbm,
                comp_v, nbr_v, idx_v, gat_v, out_v, sem):
    wid = lax.axis_index("s") * 2 + lax.axis_index("c")
    base = wid * _CHUNK
    pltpu.sync_copy(comp_hbm.at[pl.ds(base, _CHUNK)], comp_v)
    pltpu.sync_copy(nbr_hbm.at[pl.ds(base, _CHUNK)], nbr_v)
    cap = jnp.full((16,), _N - 1, jnp.int32)

    def clampstep(k, carry):
        idx_v[pl.ds(k * 16, 16)] = jnp.minimum(comp_v[pl.ds(k * 16, 16)], cap)
        return carry

    lax.fori_loop(0, _CHUNK // 16, clampstep, 0)
    cps = [pltpu.async_copy(rank_hbm.at[idx_v.at[pl.ds(j * 128, 128)]],
                            gat_v.at[pl.ds(j * 128, 128)], sem)
           for j in range(_CHUNK // 128)]
    for cp in cps:
        cp.wait()
    nval = jnp.full((16,), _N, jnp.int32)
    neg1 = jnp.full((16,), -1, jnp.int32)

    def selstep(k, carry):
        nb = nbr_v[pl.ds(k * 16, 16)]
        lbl = gat_v[pl.ds(k * 16, 16)]
        out_v[pl.ds(k * 16, 16)] = jnp.where(nb < nval, lbl, neg1)
        return carry

    lax.fori_loop(0, _CHUNK // 16, selstep, 0)
    pltpu.sync_copy(out_v, out_hbm.at[pl.ds(base, _CHUNK)])


_label = pl.kernel(
    _label_body,
    mesh=_SC_MESH,
    out_type=jax.ShapeDtypeStruct((_N,), jnp.int32),
    scratch_types=[pltpu.VMEM((_CHUNK,), jnp.int32),
                   pltpu.VMEM((_CHUNK,), jnp.int32),
                   pltpu.VMEM((_CHUNK,), jnp.int32),
                   pltpu.VMEM((_CHUNK,), jnp.int32),
                   pltpu.VMEM((_CHUNK,), jnp.int32),
                   pltpu.SemaphoreType.DMA],
)


def kernel(x):
    xh, xl = _normalize(x)
    mask, nbr3 = _build_mask(xh, xl)
    comp1 = nbr3.reshape(_N)
    comp = _jump(comp1)

    def body(st):
        t, comp, _ = st
        c2 = _jump(_sweep(mask, comp))
        return t + 1, c2, jnp.any(c2 != comp)

    def cond(st):
        return st[2] & (st[0] < 16)

    _, comp, _ = lax.while_loop(cond, body,
                                (jnp.int32(1), comp, jnp.bool_(True)))
    rank = _rank(comp)
    return _label(comp, rank, comp1)
